# baseline (speedup 1.0000x reference)
import functools

import jax
import jax.numpy as jnp
from jax import lax
from jax.experimental import pallas as pl
from jax.experimental.pallas import tpu as pltpu
from jax.experimental.pallas import tpu_sc as plsc

_VOCAB = 128
_D = 32
_NC = 2
_NS = 16
_NW = _NC * _NS
_L = 16
_R = 128
_CR = 8
_VP = _VOCAB + 8


def _bcast_lane(vec, u):
  return lax.gather(
      vec, jnp.full((_L, 1), u, jnp.int32),
      dimension_numbers=lax.GatherDimensionNumbers(
          offset_dims=(), collapsed_slice_dims=(0,), start_index_map=(0,)),
      slice_sizes=(1,),
      mode=lax.GatherScatterMode.PROMISE_IN_BOUNDS)


@functools.lru_cache(maxsize=None)
def _make_kernel(nrows: int):
  rows_w = nrows // _NW
  nch = rows_w // _CR
  assert nrows % _NW == 0 and rows_w % _CR == 0 and nch % 2 == 0

  mesh = plsc.VectorSubcoreMesh(core_axis_name="c", subcore_axis_name="s")

  @functools.partial(
      pl.kernel,
      out_type=jax.ShapeDtypeStruct((nrows, _R, _D), jnp.float32),
      mesh=mesh,
      compiler_params=pltpu.CompilerParams(
          needs_layout_passes=False, use_tc_tiling_on_sc=False),
      scratch_types=[
          pltpu.VMEM((_D * _VP,), jnp.float32),
          pltpu.VMEM((_CR, _R), jnp.int32),
          pltpu.VMEM((_CR, _R), jnp.int32),
          pltpu.VMEM((_CR, _R, _D), jnp.float32),
          pltpu.VMEM((_CR, _R, _D), jnp.float32),
          pltpu.SemaphoreType.DMA,
          pltpu.SemaphoreType.DMA,
          pltpu.SemaphoreType.DMA,
          pltpu.SemaphoreType.DMA,
      ],
  )
  def emb(x_hbm, tt_hbm, out_hbm,
          table_t, iv0, iv1, cv0, cv1, si0, si1, so0, so1):
    wid = lax.axis_index("s") * _NC + lax.axis_index("c")
    w_base = wid * rows_w
    ivs = (iv0, iv1)
    cvs = (cv0, cv1)
    sis = (si0, si1)
    sos = (so0, so1)

    pltpu.sync_copy(tt_hbm, table_t)

    for b in range(2):
      pltpu.async_copy(
          x_hbm.at[pl.ds(w_base + b * _CR, _CR)], ivs[b], sis[b])

    lanes = lax.broadcasted_iota(jnp.int32, (_L,), 0)
    c_lo = lanes * _VP
    c_hi = (lanes + _L) * _VP

    def outer(gi, carry):
      for b in range(2):
        g = gi * 2 + b
        iv, cv, si, so = ivs[b], cvs[b], sis[b], sos[b]
        base = w_base + g * _CR

        pltpu.make_async_copy(x_hbm.at[pl.ds(w_base, _CR)], iv, si).wait()

        @pl.when(gi > 0)
        def _():
          pltpu.make_async_copy(
              cv, out_hbm.at[pl.ds(w_base, _CR)], so).wait()

        for rr in range(_CR):
          ivr = iv.at[rr]

          def cgroup(j, c2, ivr=ivr, rr=rr):
            rows = ivr[pl.ds(j * _L, _L)]
            for u in range(_L):
              bc = _bcast_lane(rows, u)
              c = j * _L + u
              cv[rr, c, pl.ds(0, _L)] = plsc.load_gather(
                  table_t, [c_lo + bc])
              cv[rr, c, pl.ds(_L, _L)] = plsc.load_gather(
                  table_t, [c_hi + bc])
            return c2

          lax.fori_loop(0, _R // _L, cgroup, 0, unroll=False)

        nxt = jnp.minimum(g + 2, nch - 1)
        pltpu.async_copy(x_hbm.at[pl.ds(w_base + nxt * _CR, _CR)], iv, si)
        pltpu.async_copy(cv, out_hbm.at[pl.ds(base, _CR)], so)
      return carry

    lax.fori_loop(0, nch // 2, outer, 0, unroll=False)

    for b in range(2):
      pltpu.make_async_copy(
          x_hbm.at[pl.ds(w_base, _CR)], ivs[b], sis[b]).wait()
      pltpu.make_async_copy(
          cvs[b], out_hbm.at[pl.ds(w_base, _CR)], sos[b]).wait()

  return emb


def kernel(x, table):
  xf = x.reshape(-1, _R).astype(jnp.int32)
  tt = jnp.pad(table.T, ((0, 0), (0, _VP - _VOCAB))).reshape(-1)
  out = _make_kernel(xf.shape[0])(xf, tt)
  return out.reshape(*x.shape, _D)


# speedup vs baseline: 1.5299x; 1.5299x over previous
import functools

import jax
import jax.numpy as jnp
from jax import lax
from jax.experimental import pallas as pl
from jax.experimental.pallas import tpu as pltpu
from jax.experimental.pallas import tpu_sc as plsc

_VOCAB = 128
_D = 32
_NC = 2
_NS = 16
_NW = _NC * _NS
_L = 16
_R = 128
_CR = 8
_VP = _VOCAB + 8


def _bcast_lane(vec, u):
  return lax.gather(
      vec, jnp.full((_L, 1), u, jnp.int32),
      dimension_numbers=lax.GatherDimensionNumbers(
          offset_dims=(), collapsed_slice_dims=(0,), start_index_map=(0,)),
      slice_sizes=(1,),
      mode=lax.GatherScatterMode.PROMISE_IN_BOUNDS)


@functools.lru_cache(maxsize=None)
def _make_kernel(nrows: int):
  rows_w = nrows // _NW
  nch = rows_w // _CR
  assert nrows % _NW == 0 and rows_w % _CR == 0 and nch % 2 == 0

  mesh = plsc.VectorSubcoreMesh(core_axis_name="c", subcore_axis_name="s")

  @functools.partial(
      pl.kernel,
      out_type=jax.ShapeDtypeStruct((nrows, _R, _D), jnp.float32),
      mesh=mesh,
      compiler_params=pltpu.CompilerParams(
          needs_layout_passes=False, use_tc_tiling_on_sc=False),
      scratch_types=[
          pltpu.VMEM((_D * _VP,), jnp.float32),
          pltpu.VMEM((_CR, _R), jnp.int32),
          pltpu.VMEM((_CR, _R), jnp.int32),
          pltpu.VMEM((_CR, _R, _D), jnp.float32),
          pltpu.VMEM((_CR, _R, _D), jnp.float32),
          pltpu.SemaphoreType.DMA,
          pltpu.SemaphoreType.DMA,
          pltpu.SemaphoreType.DMA,
          pltpu.SemaphoreType.DMA,
      ],
  )
  def emb(x_hbm, tt_hbm, out_hbm,
          table_t, iv0, iv1, cv0, cv1, si0, si1, so0, so1):
    wid = lax.axis_index("s") * _NC + lax.axis_index("c")
    w_base = wid * rows_w
    ivs = (iv0, iv1)
    cvs = (cv0, cv1)
    sis = (si0, si1)
    sos = (so0, so1)

    pltpu.sync_copy(tt_hbm, table_t)

    for b in range(2):
      pltpu.async_copy(
          x_hbm.at[pl.ds(w_base + b * _CR, _CR)], ivs[b], sis[b])

    lanes = lax.broadcasted_iota(jnp.int32, (_L,), 0)
    c_lo = lanes * _VP
    c_hi = (lanes + _L) * _VP

    def outer(gi, carry):
      for b in range(2):
        g = gi * 2 + b
        iv, cv, si, so = ivs[b], cvs[b], sis[b], sos[b]
        base = w_base + g * _CR

        pltpu.make_async_copy(x_hbm.at[pl.ds(w_base, _CR)], iv, si).wait()

        @pl.when(gi > 0)
        def _():
          pltpu.make_async_copy(
              cv, out_hbm.at[pl.ds(w_base, _CR)], so).wait()

        for rr in range(_CR):
          ivr = iv.at[rr]

          @functools.partial(
              plsc.parallel_loop, 0, _R // _L, unroll=2)
          def cgroup(j, ivr=ivr, rr=rr):
            rows = ivr[pl.ds(j * _L, _L)]
            for u in range(_L):
              bc = _bcast_lane(rows, u)
              c = j * _L + u
              cv[rr, c, pl.ds(0, _L)] = plsc.load_gather(
                  table_t, [c_lo + bc])
              cv[rr, c, pl.ds(_L, _L)] = plsc.load_gather(
                  table_t, [c_hi + bc])

        nxt = jnp.minimum(g + 2, nch - 1)
        pltpu.async_copy(x_hbm.at[pl.ds(w_base + nxt * _CR, _CR)], iv, si)
        pltpu.async_copy(cv, out_hbm.at[pl.ds(base, _CR)], so)
      return carry

    lax.fori_loop(0, nch // 2, outer, 0, unroll=False)

    for b in range(2):
      pltpu.make_async_copy(
          x_hbm.at[pl.ds(w_base, _CR)], ivs[b], sis[b]).wait()
      pltpu.make_async_copy(
          cvs[b], out_hbm.at[pl.ds(w_base, _CR)], sos[b]).wait()

  return emb


def kernel(x, table):
  xf = x.reshape(-1, _R).astype(jnp.int32)
  tt = jnp.pad(table.T, ((0, 0), (0, _VP - _VOCAB))).reshape(-1)
  out = _make_kernel(xf.shape[0])(xf, tt)
  return out.reshape(*x.shape, _D)
